# Initial kernel scaffold; baseline (speedup 1.0000x reference)
import jax
import jax.numpy as jnp
from jax.experimental import pallas as pl


def kernel(out_image, segment_image, batch_num):
    raise NotImplementedError("write your pallas kernel here")


# speedup vs baseline: 9.4985x; 9.4985x over previous
import jax
import jax.numpy as jnp
from jax.experimental import pallas as pl
from jax.experimental.pallas import tpu as pltpu


def _shift_rows(x, di):
    n = x.shape[0]
    r = jnp.roll(x, -di, axis=0)
    ii = jax.lax.broadcasted_iota(jnp.int32, x.shape, 0)
    if di > 0:
        return jnp.where(ii < n - di, r, 0.0)
    return jnp.where(ii >= -di, r, 0.0)


def _shift_cols(x, dj):
    n = x.shape[1]
    r = jnp.roll(x, -dj, axis=1)
    jj = jax.lax.broadcasted_iota(jnp.int32, x.shape, 1)
    if dj > 0:
        return jnp.where(jj < n - dj, r, 0.0)
    return jnp.where(jj >= -dj, r, 0.0)


def _nbsum(x):
    row = x + _shift_cols(x, 1) + _shift_cols(x, -1)
    return row + _shift_rows(row, 1) + _shift_rows(row, -1) - x


def _loss_kernel(m_ref, s_ref, out_ref, acc_ref):
    p = pl.program_id(0)
    m = m_ref[0]
    s = s_ref[0]
    h, w = m.shape

    ii = jax.lax.broadcasted_iota(jnp.int32, (h, w), 0)
    jj = jax.lax.broadcasted_iota(jnp.int32, (h, w), 1)
    kill = ((ii == h - 1) & (jj >= 1)) | ((jj == w - 1) & (ii >= 1))
    keep = jnp.where(kill, 0.0, 1.0)

    sum_t = _nbsum(s) * keep
    u = jnp.where(s != 0.0, sum_t, 0.0)
    v = u * jnp.abs(s - m)
    den = _nbsum(u)
    num = _nbsum(v)
    pix = (num / (den + 1.0)) * keep

    prom = m >= 0.8
    s_sum = jnp.sum(jnp.where(prom, pix, 0.0))
    cnt = jnp.sum(prom.astype(jnp.float32))

    prev = jnp.where(p == 0, 0.0, acc_ref[0])
    new = (prev + s_sum) / cnt
    acc_ref[0] = new

    @pl.when(p == pl.num_programs(0) - 1)
    def _():
        out_ref[0] = new


def kernel(out_image, segment_image, batch_num):
    b, _, h, w = out_image.shape
    m = out_image.reshape(b, h, w)
    s = segment_image.reshape(b, h, w)
    out = pl.pallas_call(
        _loss_kernel,
        grid=(b,),
        in_specs=[
            pl.BlockSpec((1, h, w), lambda p: (p, 0, 0)),
            pl.BlockSpec((1, h, w), lambda p: (p, 0, 0)),
        ],
        out_specs=pl.BlockSpec(memory_space=pltpu.SMEM),
        out_shape=jax.ShapeDtypeStruct((1,), jnp.float32),
        scratch_shapes=[pltpu.SMEM((1,), jnp.float32)],
    )(m, s)
    return out[0] / batch_num
